# baseline (speedup 1.0000x reference)
import functools

import jax
import jax.numpy as jnp
from jax import lax
from jax.experimental import pallas as pl
from jax.experimental.pallas import tpu as pltpu
from jax.experimental.pallas import tpu_sc as plsc

NC = 2
NS = 16
LANES = 16
NW = NC * NS

VOCAB = 1000000
BATCH = 4096
SEQ = 200
DIM = 64
TOTAL = SEQ * BATCH
CHUNK = 128
BCHUNKS = BATCH // CHUNK
NBLOCKS = SEQ * BCHUNKS
BL_PER_W = NBLOCKS // NW
KB = DIM // 8
PRE_T = 4096


def _eye(n):
    r = lax.broadcasted_iota(jnp.int32, (n, n), 0)
    c = lax.broadcasted_iota(jnp.int32, (n, n), 1)
    return (r == c).astype(jnp.float32)


def _mxu_t(x):
    return lax.dot_general(x, _eye(x.shape[0]), (((0,), (0,)), ((), ())),
                           preferred_element_type=jnp.float32)


def _pre_body(t_ref, o_ref):
    t = _mxu_t(t_ref[...])
    parts = []
    for grp in range(PRE_T // 512):
        lo = t[grp * 512:grp * 512 + 256]
        hi = t[grp * 512 + 256:(grp + 1) * 512]
        parts.append(jnp.concatenate([lo, hi], axis=1))
    o_ref[...] = jnp.concatenate(parts, axis=0)


def _mid_body(idx_hbm, tab_hbm, x2_hbm, idx_v, h_v, g_v, sem_g, sem_o):
    wid = lax.axis_index("s") * NC + lax.axis_index("c")
    blk0 = wid * BL_PER_W

    pltpu.sync_copy(idx_hbm.at[pl.ds(blk0 * CHUNK, BL_PER_W * CHUNK)], idx_v)

    def build_and_fire(m, hb, gb):
        for g in range(CHUNK // LANES):
            iv = idx_v[pl.ds(m * CHUNK + g * LANES, LANES)]
            hv = lax.shift_left(lax.shift_right_logical(iv, 9),
                                jnp.full((LANES,), 8, jnp.int32)) + \
                 (iv & jnp.full((LANES,), 255, jnp.int32))
            h_v.at[hb, pl.ds(g * LANES, LANES)][...] = hv
        pltpu.async_copy(tab_hbm.at[h_v.at[hb]], g_v.at[gb], sem_g)

    def wait_g(b):
        pltpu.make_async_copy(tab_hbm.at[h_v.at[b]], g_v.at[b], sem_g).wait()

    def drain_x(b):
        pltpu.make_async_copy(g_v.at[b], x2_hbm.at[pl.ds(0, CHUNK)], sem_o).wait()

    NB = 4
    build_and_fire(0, 0, 0)
    build_and_fire(1, 1, 1)

    def step(it, carry):
        for sub in range(NB):
            m = it * NB + sub
            b = sub
            b2 = (sub + 2) % NB

            @pl.when(m >= 2)
            def _():
                drain_x(b2)

            @pl.when(m + 2 < BL_PER_W)
            def _():
                build_and_fire(m + 2, b2, b2)

            wait_g(b)
            pltpu.async_copy(
                g_v.at[b], x2_hbm.at[pl.ds((blk0 + m) * CHUNK, CHUNK)], sem_o)
        return carry

    lax.fori_loop(0, BL_PER_W // NB, step, 0)
    drain_x((BL_PER_W - 2) % NB)
    drain_x((BL_PER_W - 1) % NB)


def _post_body(x_ref, i_ref, p_ref, o_ref):
    g = x_ref[0]
    par = (i_ref[0] >> 8) & 1
    sel = jnp.where(par == 1, g[:, DIM:], g[:, :DIM])
    y = lax.dot_general(_eye(DIM), sel, (((1,), (1,)), ((), ())),
                        preferred_element_type=jnp.float32)
    y = y + p_ref[0]
    for k in range(KB):
        for c in range(BCHUNKS):
            o_ref[0, k, c] = y[k * 8:(k + 1) * 8, c * CHUNK:(c + 1) * CHUNK]


@jax.jit
def kernel(inputs, token_table, position_table):
    tab_t = token_table.T
    idx_flat = inputs.T.reshape(TOTAL)

    grid_pre = VOCAB // PRE_T + 1
    tab2 = pl.pallas_call(
        _pre_body,
        grid=(grid_pre,),
        in_specs=[pl.BlockSpec((DIM, PRE_T), lambda j: (0, j))],
        out_specs=pl.BlockSpec((PRE_T // 2, 2 * DIM), lambda j: (j, 0)),
        out_shape=jax.ShapeDtypeStruct((grid_pre * PRE_T // 2, 2 * DIM),
                                       jnp.float32),
    )(tab_t)

    mesh = plsc.VectorSubcoreMesh(core_axis_name="c", subcore_axis_name="s")
    run_mid = functools.partial(
        pl.kernel,
        out_type=jax.ShapeDtypeStruct((TOTAL, 2 * DIM), jnp.float32),
        mesh=mesh,
        scratch_types=[
            pltpu.VMEM((BL_PER_W * CHUNK,), jnp.int32),
            pltpu.VMEM((4, CHUNK), jnp.int32),
            pltpu.VMEM((4, CHUNK, 2 * DIM), jnp.float32),
            pltpu.SemaphoreType.DMA,
            pltpu.SemaphoreType.DMA,
        ],
        compiler_params=pltpu.CompilerParams(use_tc_tiling_on_sc=False,
                                             needs_layout_passes=False),
    )(_mid_body)
    x2 = run_mid(idx_flat, tab2)

    out5 = pl.pallas_call(
        _post_body,
        grid=(SEQ,),
        in_specs=[
            pl.BlockSpec((1, BATCH, 2 * DIM), lambda s: (s, 0, 0)),
            pl.BlockSpec((1, BATCH, 1), lambda s: (s, 0, 0)),
            pl.BlockSpec((1, DIM, 1), lambda s: (s, 0, 0)),
        ],
        out_specs=pl.BlockSpec((1, KB, BCHUNKS, 8, CHUNK),
                               lambda s: (s, 0, 0, 0, 0)),
        out_shape=jax.ShapeDtypeStruct((SEQ, KB, BCHUNKS, 8, CHUNK), jnp.float32),
    )(x2.reshape(SEQ, BATCH, 2 * DIM),
      idx_flat.reshape(SEQ, BATCH, 1),
      position_table.reshape(SEQ, DIM, 1))

    return out5.transpose(2, 4, 0, 1, 3).reshape(BATCH, SEQ, DIM)


# speedup vs baseline: 1.2166x; 1.2166x over previous
import functools

import jax
import jax.numpy as jnp
from jax import lax
from jax.experimental import pallas as pl
from jax.experimental.pallas import tpu as pltpu
from jax.experimental.pallas import tpu_sc as plsc

NC = 2
NS = 16
LANES = 16
NW = NC * NS

VOCAB = 1000000
BATCH = 4096
SEQ = 200
DIM = 64
TOTAL = SEQ * BATCH
CHUNK = 128
BCHUNKS = BATCH // CHUNK
NBLOCKS = SEQ * BCHUNKS
BL_PER_W = NBLOCKS // NW
KB = DIM // 8
PRE_T = 4096


def _eye(n):
    r = lax.broadcasted_iota(jnp.int32, (n, n), 0)
    c = lax.broadcasted_iota(jnp.int32, (n, n), 1)
    return (r == c).astype(jnp.float32)


def _mxu_t(x):
    return lax.dot_general(x, _eye(x.shape[0]), (((0,), (0,)), ((), ())),
                           preferred_element_type=jnp.float32)


def _pre_body(t_ref, o_ref):
    t = _mxu_t(t_ref[...])
    parts = []
    for grp in range(PRE_T // 512):
        lo = t[grp * 512:grp * 512 + 256]
        hi = t[grp * 512 + 256:(grp + 1) * 512]
        parts.append(jnp.concatenate([lo, hi], axis=1))
    o_ref[...] = jnp.concatenate(parts, axis=0)


def _mid_body(idx_hbm, tab_hbm, x2_hbm, idx_v, h_v, g_v, sem_g, sem_o):
    wid = lax.axis_index("s") * NC + lax.axis_index("c")
    blk0 = wid * BL_PER_W

    pltpu.sync_copy(idx_hbm.at[pl.ds(blk0 * CHUNK, BL_PER_W * CHUNK)], idx_v)

    def build_and_fire(m, hb, gb):
        for g in range(CHUNK // LANES):
            iv = idx_v[pl.ds(m * CHUNK + g * LANES, LANES)]
            hv = lax.shift_left(lax.shift_right_logical(iv, 9),
                                jnp.full((LANES,), 8, jnp.int32)) + \
                 (iv & jnp.full((LANES,), 255, jnp.int32))
            h_v.at[hb, pl.ds(g * LANES, LANES)][...] = hv
        pltpu.async_copy(tab_hbm.at[h_v.at[hb]], g_v.at[gb], sem_g)

    def wait_g(b):
        pltpu.make_async_copy(tab_hbm.at[h_v.at[b]], g_v.at[b], sem_g).wait()

    def drain_x(b):
        pltpu.make_async_copy(g_v.at[b], x2_hbm.at[pl.ds(0, CHUNK)], sem_o).wait()

    NB = 4
    build_and_fire(0, 0, 0)
    build_and_fire(1, 1, 1)

    def step(it, carry):
        for sub in range(NB):
            m = it * NB + sub
            b = sub
            b2 = (sub + 2) % NB

            @pl.when(m >= 2)
            def _():
                drain_x(b2)

            @pl.when(m + 2 < BL_PER_W)
            def _():
                build_and_fire(m + 2, b2, b2)

            wait_g(b)
            pltpu.async_copy(
                g_v.at[b], x2_hbm.at[pl.ds((blk0 + m) * CHUNK, CHUNK)], sem_o)
        return carry

    lax.fori_loop(0, BL_PER_W // NB, step, 0)
    drain_x((BL_PER_W - 2) % NB)
    drain_x((BL_PER_W - 1) % NB)


def _post_body(x_ref, i_ref, p_ref, o_ref):
    g = x_ref[0]
    gt = lax.dot_general(_eye(2 * DIM), g, (((1,), (1,)), ((), ())),
                         preferred_element_type=jnp.float32)
    par = (i_ref[0, 0] >> 8) & 1
    sel = jnp.where(par == 1, gt[DIM:], gt[:DIM])
    y = sel + p_ref[0]
    for k in range(KB):
        for c in range(BCHUNKS):
            o_ref[0, k, c] = y[k * 8:(k + 1) * 8, c * CHUNK:(c + 1) * CHUNK]


@jax.jit
def kernel(inputs, token_table, position_table):
    tab_t = token_table.T
    idx_flat = inputs.T.reshape(TOTAL)

    grid_pre = VOCAB // PRE_T + 1
    tab2 = pl.pallas_call(
        _pre_body,
        grid=(grid_pre,),
        in_specs=[pl.BlockSpec((DIM, PRE_T), lambda j: (0, j))],
        out_specs=pl.BlockSpec((PRE_T // 2, 2 * DIM), lambda j: (j, 0)),
        out_shape=jax.ShapeDtypeStruct((grid_pre * PRE_T // 2, 2 * DIM),
                                       jnp.float32),
    )(tab_t)

    mesh = plsc.VectorSubcoreMesh(core_axis_name="c", subcore_axis_name="s")
    run_mid = functools.partial(
        pl.kernel,
        out_type=jax.ShapeDtypeStruct((TOTAL, 2 * DIM), jnp.float32),
        mesh=mesh,
        scratch_types=[
            pltpu.VMEM((BL_PER_W * CHUNK,), jnp.int32),
            pltpu.VMEM((4, CHUNK), jnp.int32),
            pltpu.VMEM((4, CHUNK, 2 * DIM), jnp.float32),
            pltpu.SemaphoreType.DMA,
            pltpu.SemaphoreType.DMA,
        ],
        compiler_params=pltpu.CompilerParams(use_tc_tiling_on_sc=False,
                                             needs_layout_passes=False),
    )(_mid_body)
    x2 = run_mid(idx_flat, tab2)

    out5 = pl.pallas_call(
        _post_body,
        grid=(SEQ,),
        in_specs=[
            pl.BlockSpec((1, BATCH, 2 * DIM), lambda s: (s, 0, 0)),
            pl.BlockSpec((1, 1, BATCH), lambda s: (s, 0, 0)),
            pl.BlockSpec((1, DIM, 1), lambda s: (s, 0, 0)),
        ],
        out_specs=pl.BlockSpec((1, KB, BCHUNKS, 8, CHUNK),
                               lambda s: (s, 0, 0, 0, 0)),
        out_shape=jax.ShapeDtypeStruct((SEQ, KB, BCHUNKS, 8, CHUNK), jnp.float32),
    )(x2.reshape(SEQ, BATCH, 2 * DIM),
      idx_flat.reshape(SEQ, 1, BATCH),
      position_table.reshape(SEQ, DIM, 1))

    return out5.transpose(2, 4, 0, 1, 3).reshape(BATCH, SEQ, DIM)
